# baseline (speedup 1.0000x reference)
import jax
import jax.numpy as jnp
from jax import lax
from jax.experimental import pallas as pl
from jax.experimental.pallas import tpu as pltpu
from jax.experimental.pallas import tpu_sc as plsc

B = 16384
NH = 24
DH = 8
ND = 7
DD = 4
D = DH + DD
NC = 1
NS = 16
NW = NC * NS
BPW = B // NW
L = 16


def _emb_body(hour_hbm, dow_hbm, ht_hbm, dt_hbm, out_hbm,
              hour_v, dow_v, ht_v, dt_v, out_v, sem):
    wid = lax.axis_index("s") * NC + lax.axis_index("c")
    base = wid * BPW
    copies = [
        pltpu.async_copy(ht_hbm, ht_v, sem),
        pltpu.async_copy(dt_hbm, dt_v, sem),
        pltpu.async_copy(hour_hbm.at[pl.ds(base, BPW)], hour_v, sem),
        pltpu.async_copy(dow_hbm.at[pl.ds(base, BPW)], dow_v, sem),
    ]
    for c in copies:
        c.wait()
    cvec = [jnp.full((L,), c, jnp.int32) for c in range(DH)]
    for g in range(BPW // L):
        sl = pl.ds(g * L, L)
        hv = hour_v[sl]
        dv = dow_v[sl]
        for c in range(DH):
            out_v[c, sl] = plsc.load_gather(ht_v, [cvec[c], hv])
        for c in range(DD):
            out_v[DH + c, sl] = plsc.load_gather(dt_v, [cvec[c], dv])
    pltpu.sync_copy(out_v, out_hbm.at[:, pl.ds(base, BPW)])


@jax.jit
def _lookup(hour, dow, ht_t, dt_t):
    mesh = plsc.VectorSubcoreMesh(core_axis_name="c", subcore_axis_name="s",
                                  num_cores=NC)
    f = pl.kernel(
        _emb_body,
        out_type=jax.ShapeDtypeStruct((D, B), jnp.float32),
        mesh=mesh,
        compiler_params=pltpu.CompilerParams(needs_layout_passes=False),
        scratch_types=[
            pltpu.VMEM((BPW,), jnp.int32),
            pltpu.VMEM((BPW,), jnp.int32),
            pltpu.VMEM((DH, NH), jnp.float32),
            pltpu.VMEM((DD, ND), jnp.float32),
            pltpu.VMEM((D, BPW), jnp.float32),
            pltpu.SemaphoreType.DMA,
        ],
    )
    return f(hour, dow, ht_t, dt_t)


def kernel(hour, dow, dom, hour_table, dow_table):
    del dom
    out_t = _lookup(hour.astype(jnp.int32), dow.astype(jnp.int32),
                    hour_table.T, dow_table.T)
    return out_t.T


# speedup vs baseline: 1.2051x; 1.2051x over previous
import jax
import jax.numpy as jnp
from jax import lax
from jax.experimental import pallas as pl
from jax.experimental.pallas import tpu as pltpu
from jax.experimental.pallas import tpu_sc as plsc

B = 16384
NH = 24
DH = 8
ND = 7
DD = 4
D = DH + DD
NC = 2
NS = 16
NW = NC * NS
BPW = B // NW
L = 16


def _emb_body(hour_hbm, dow_hbm, ht_hbm, dt_hbm, out_hbm,
              hour_v, dow_v, ht_v, dt_v, out_v, sem):
    wid = lax.axis_index("s") * NC + lax.axis_index("c")
    base = wid * BPW
    copies = [
        pltpu.async_copy(ht_hbm, ht_v, sem),
        pltpu.async_copy(dt_hbm, dt_v, sem),
        pltpu.async_copy(hour_hbm.at[pl.ds(base, BPW)], hour_v, sem),
        pltpu.async_copy(dow_hbm.at[pl.ds(base, BPW)], dow_v, sem),
    ]
    for c in copies:
        c.wait()
    cvec = [jnp.full((L,), c, jnp.int32) for c in range(DH)]
    for g in range(BPW // L):
        sl = pl.ds(g * L, L)
        hv = hour_v[sl]
        dv = dow_v[sl]
        vals = [plsc.load_gather(ht_v, [cvec[c], hv]) for c in range(DH)]
        vals += [plsc.load_gather(dt_v, [cvec[c], dv]) for c in range(DD)]
        for c in range(D):
            out_v[c, sl] = vals[c]
    pltpu.sync_copy(out_v, out_hbm.at[:, pl.ds(base, BPW)])


@jax.jit
def _lookup(hour, dow, ht_t, dt_t):
    mesh = plsc.VectorSubcoreMesh(core_axis_name="c", subcore_axis_name="s",
                                  num_cores=NC)
    f = pl.kernel(
        _emb_body,
        out_type=jax.ShapeDtypeStruct((D, B), jnp.float32),
        mesh=mesh,
        compiler_params=pltpu.CompilerParams(needs_layout_passes=False),
        scratch_types=[
            pltpu.VMEM((BPW,), jnp.int32),
            pltpu.VMEM((BPW,), jnp.int32),
            pltpu.VMEM((DH, NH), jnp.float32),
            pltpu.VMEM((DD, ND), jnp.float32),
            pltpu.VMEM((D, BPW), jnp.float32),
            pltpu.SemaphoreType.DMA,
        ],
    )
    return f(hour, dow, ht_t, dt_t)


def kernel(hour, dow, dom, hour_table, dow_table):
    del dom
    out_t = _lookup(hour.astype(jnp.int32), dow.astype(jnp.int32),
                    hour_table.T, dow_table.T)
    return out_t.T


# speedup vs baseline: 1.2635x; 1.0484x over previous
import jax
import jax.numpy as jnp
from jax import lax
from jax.experimental import pallas as pl
from jax.experimental.pallas import tpu as pltpu
from jax.experimental.pallas import tpu_sc as plsc

B = 16384
NH = 24
DH = 8
ND = 7
DD = 4
D = DH + DD
NC = 2
NS = 16
NW = NC * NS
BPW = B // NW
L = 16


def _emb_body(hour_hbm, dow_hbm, ht_hbm, dt_hbm, out_hbm,
              hour_v, dow_v, ht_v, dt_v, out_v, sem):
    wid = lax.axis_index("s") * NC + lax.axis_index("c")
    base = wid * BPW
    copies = [
        pltpu.async_copy(ht_hbm, ht_v, sem),
        pltpu.async_copy(dt_hbm, dt_v, sem),
        pltpu.async_copy(hour_hbm.at[pl.ds(base, BPW)], hour_v, sem),
        pltpu.async_copy(dow_hbm.at[pl.ds(base, BPW)], dow_v, sem),
    ]
    for c in copies:
        c.wait()
    cvec = [jnp.full((L,), c, jnp.int32) for c in range(DH)]

    @plsc.parallel_loop(0, BPW // L, unroll=2)
    def _group(g):
        sl = pl.ds(g * L, L)
        hv = hour_v[sl]
        dv = dow_v[sl]
        vals = [plsc.load_gather(ht_v, [cvec[c], hv]) for c in range(DH)]
        vals += [plsc.load_gather(dt_v, [cvec[c], dv]) for c in range(DD)]
        for c in range(D):
            out_v[c, sl] = vals[c]
    pltpu.sync_copy(out_v, out_hbm.at[:, pl.ds(base, BPW)])


@jax.jit
def _lookup(hour, dow, ht_t, dt_t):
    mesh = plsc.VectorSubcoreMesh(core_axis_name="c", subcore_axis_name="s",
                                  num_cores=NC)
    f = pl.kernel(
        _emb_body,
        out_type=jax.ShapeDtypeStruct((D, B), jnp.float32),
        mesh=mesh,
        compiler_params=pltpu.CompilerParams(needs_layout_passes=False),
        scratch_types=[
            pltpu.VMEM((BPW,), jnp.int32),
            pltpu.VMEM((BPW,), jnp.int32),
            pltpu.VMEM((DH, NH), jnp.float32),
            pltpu.VMEM((DD, ND), jnp.float32),
            pltpu.VMEM((D, BPW), jnp.float32),
            pltpu.SemaphoreType.DMA,
        ],
    )
    return f(hour, dow, ht_t, dt_t)


def kernel(hour, dow, dom, hour_table, dow_table):
    del dom
    out_t = _lookup(hour.astype(jnp.int32), dow.astype(jnp.int32),
                    hour_table.T, dow_table.T)
    return out_t.T
